# Initial kernel scaffold; baseline (speedup 1.0000x reference)
import jax
import jax.numpy as jnp
from jax.experimental import pallas as pl


def kernel(x):
    raise NotImplementedError("write your pallas kernel here")


# speedup vs baseline: 3.0017x; 3.0017x over previous
import jax
import jax.numpy as jnp
from jax.experimental import pallas as pl
from jax.experimental.pallas import tpu as pltpu

_B, _N, _L = 4, 4096, 1024
_BLK = 256


def _scan_body(x_ref, o_ref, carry_ref):
    i = pl.program_id(1)

    @pl.when(i == 0)
    def _():
        carry_ref[...] = jnp.zeros_like(carry_ref)

    x = x_ref[0]
    rows = jax.lax.broadcasted_iota(jnp.int32, (_BLK, _BLK), 0)
    cols = jax.lax.broadcasted_iota(jnp.int32, (_BLK, _BLK), 1)
    tri = (cols < rows).astype(jnp.float32)
    excl = jnp.dot(tri, x, preferred_element_type=jnp.float32)
    o_ref[0] = excl + carry_ref[...]
    carry_ref[...] += jnp.sum(x, axis=0, keepdims=True)


def kernel(x):
    return pl.pallas_call(
        _scan_body,
        grid=(_B, _N // _BLK),
        in_specs=[pl.BlockSpec((1, _BLK, _L), lambda b, i: (b, i, 0))],
        out_specs=pl.BlockSpec((1, _BLK, _L), lambda b, i: (b, i, 0)),
        out_shape=jax.ShapeDtypeStruct((_B, _N, _L), jnp.float32),
        scratch_shapes=[pltpu.VMEM((1, _L), jnp.float32)],
    )(x)
